# baseline (speedup 1.0000x reference)
import jax
import jax.numpy as jnp
from jax import lax
from jax.experimental import pallas as pl
from jax.experimental.pallas import tpu as pltpu
from jax.experimental.pallas import tpu_sc as plsc

N = 10000
NPAD = 10240
E = 320000
D = 128

NC = 2
NS = 16
NW = NC * NS

EPW = E // NW
CHUNK = 112
NFULL = EPW // CHUNK
TAIL = EPW - NFULL * CHUNK
RPT = NPAD // NS
RB = 80
NRB = RPT // RB

BR = 2000

_mesh = plsc.VectorSubcoreMesh(core_axis_name="c", subcore_axis_name="s")


def _hist_body(dst_hbm, ones_hbm, zeros_hbm, out_hbm, dst_v, ones_v,
               hist_sh, sem):
  cid = lax.axis_index("c")
  sid = lax.axis_index("s")
  wid = cid * NS + sid

  pltpu.sync_copy(ones_hbm, ones_v)

  @pl.loop(0, NRB)
  def _(k):
    pltpu.sync_copy(zeros_hbm, hist_sh.at[pl.ds(sid * RPT + k * RB, RB)])

  pltpu.sync_copy(dst_hbm.at[wid, 0], dst_v)
  plsc.subcore_barrier()

  @pl.loop(0, NFULL)
  def _(j):
    pltpu.sync_copy(ones_v, hist_sh.at[dst_v.at[pl.ds(j * CHUNK, CHUNK)]],
                    add=True)

  pltpu.sync_copy(ones_v.at[pl.ds(0, TAIL)],
                  hist_sh.at[dst_v.at[pl.ds(NFULL * CHUNK, TAIL)]], add=True)

  plsc.subcore_barrier()

  @pl.loop(0, NRB)
  def _(k):
    sl = pl.ds(sid * RPT + k * RB, RB)
    pltpu.sync_copy(hist_sh.at[sl], out_hbm.at[cid, sl])


def _sc_histogram(dst_idx, ones_blk, zeros_rb):
  return pl.kernel(
      _hist_body,
      out_type=jax.ShapeDtypeStruct((NC, NPAD, D), jnp.float32),
      mesh=_mesh,
      scratch_types=[
          pltpu.VMEM((EPW,), jnp.int32),
          pltpu.VMEM((CHUNK, D), jnp.float32),
          pltpu.VMEM_SHARED((NPAD, D), jnp.float32),
          pltpu.SemaphoreType.DMA,
      ],
  )(dst_idx, ones_blk, zeros_rb)


def _scat_body(h_hbm, src_hbm, dst_hbm, zeros_hbm, out_hbm, src_v, dst_v,
               rows_v, acc_sh, gsem0, gsem1, ssem0, ssem1):
  cid = lax.axis_index("c")
  sid = lax.axis_index("s")
  wid = cid * NS + sid

  @pl.loop(0, NRB)
  def _(k):
    pltpu.sync_copy(zeros_hbm, acc_sh.at[pl.ds(sid * RPT + k * RB, RB)])

  pltpu.sync_copy(src_hbm.at[wid, 0], src_v)
  pltpu.sync_copy(dst_hbm.at[wid, 0], dst_v)
  plsc.subcore_barrier()

  def gather_start(j, p, sem):
    pltpu.async_copy(
        h_hbm.at[src_v.at[pl.ds(j * CHUNK, CHUNK)]], rows_v.at[p], sem)

  def gather_wait(p, sem):
    pltpu.make_async_copy(
        h_hbm.at[src_v.at[pl.ds(0, CHUNK)]], rows_v.at[p], sem).wait()

  def scatter_add(j, p):
    pltpu.sync_copy(
        rows_v.at[p], acc_sh.at[dst_v.at[pl.ds(j * CHUNK, CHUNK)]], add=True)

  @pl.loop(0, NFULL)
  def _(j):
    gather_start(j, 0, gsem0)
    gather_wait(0, gsem0)
    scatter_add(j, 0)

  pltpu.async_copy(
      h_hbm.at[src_v.at[pl.ds(NFULL * CHUNK, TAIL)]],
      rows_v.at[1, pl.ds(0, TAIL)], gsem1)
  pltpu.make_async_copy(
      h_hbm.at[src_v.at[pl.ds(0, TAIL)]],
      rows_v.at[1, pl.ds(0, TAIL)], gsem1).wait()
  pltpu.sync_copy(
      rows_v.at[1, pl.ds(0, TAIL)],
      acc_sh.at[dst_v.at[pl.ds(NFULL * CHUNK, TAIL)]], add=True)

  plsc.subcore_barrier()

  @pl.loop(0, NRB)
  def _(k):
    sl = pl.ds(sid * RPT + k * RB, RB)
    pltpu.sync_copy(acc_sh.at[sl], out_hbm.at[cid, sl])


def _sc_scatter(h, src_idx, dst_idx, zeros_rb):
  return pl.kernel(
      _scat_body,
      out_type=jax.ShapeDtypeStruct((NC, NPAD, D), jnp.float32),
      mesh=_mesh,
      scratch_types=[
          pltpu.VMEM((EPW,), jnp.int32),
          pltpu.VMEM((EPW,), jnp.int32),
          pltpu.VMEM((2, CHUNK, D), jnp.float32),
          pltpu.VMEM_SHARED((NPAD, D), jnp.float32),
          pltpu.SemaphoreType.DMA,
          pltpu.SemaphoreType.DMA,
          pltpu.SemaphoreType.DMA,
          pltpu.SemaphoreType.DMA,
      ],
  )(h, src_idx, dst_idx, zeros_rb)


def _dinv_from_hist(hist_blk):
  deg = hist_blk[0, :, 0] + hist_blk[1, :, 0] + 1.0
  return lax.rsqrt(deg)[:, None]


def _tc_first_body(hist_ref, x_ref, w_ref, out_ref):
  dinv = _dinv_from_hist(hist_ref[...])
  h = jnp.dot(x_ref[...], w_ref[...], preferred_element_type=jnp.float32)
  out_ref[...] = h * dinv


def _tc_first(hist, x, W):
  return pl.pallas_call(
      _tc_first_body,
      grid=(N // BR,),
      in_specs=[
          pl.BlockSpec((NC, BR, D), lambda i: (0, i, 0)),
          pl.BlockSpec((BR, D), lambda i: (i, 0)),
          pl.BlockSpec((D, D), lambda i: (0, 0)),
      ],
      out_specs=pl.BlockSpec((BR, D), lambda i: (i, 0)),
      out_shape=jax.ShapeDtypeStruct((N, D), jnp.float32),
  )(hist, x, W)


def _tc_advance_body(hist_ref, acc_ref, h_ref, b_ref, w_ref, out_ref):
  dinv = _dinv_from_hist(hist_ref[...])
  z = (acc_ref[0] + acc_ref[1] + h_ref[...]) * dinv + b_ref[...]
  xn = jnp.maximum(z, 0.0)
  out_ref[...] = jnp.dot(
      xn, w_ref[...], preferred_element_type=jnp.float32) * dinv


def _tc_advance(hist, acc, h, b, W):
  return pl.pallas_call(
      _tc_advance_body,
      grid=(N // BR,),
      in_specs=[
          pl.BlockSpec((NC, BR, D), lambda i: (0, i, 0)),
          pl.BlockSpec((NC, BR, D), lambda i: (0, i, 0)),
          pl.BlockSpec((BR, D), lambda i: (i, 0)),
          pl.BlockSpec((1, D), lambda i: (0, 0)),
          pl.BlockSpec((D, D), lambda i: (0, 0)),
      ],
      out_specs=pl.BlockSpec((BR, D), lambda i: (i, 0)),
      out_shape=jax.ShapeDtypeStruct((N, D), jnp.float32),
  )(hist, acc, h, b, W)


def _tc_final_body(hist_ref, acc_ref, h_ref, b_ref, out_ref):
  dinv = _dinv_from_hist(hist_ref[...])
  out_ref[...] = (acc_ref[0] + acc_ref[1] + h_ref[...]) * dinv + b_ref[...]


def _tc_final(hist, acc, h, b):
  return pl.pallas_call(
      _tc_final_body,
      grid=(N // BR,),
      in_specs=[
          pl.BlockSpec((NC, BR, D), lambda i: (0, i, 0)),
          pl.BlockSpec((NC, BR, D), lambda i: (0, i, 0)),
          pl.BlockSpec((BR, D), lambda i: (i, 0)),
          pl.BlockSpec((1, D), lambda i: (0, 0)),
      ],
      out_specs=pl.BlockSpec((BR, D), lambda i: (i, 0)),
      out_shape=jax.ShapeDtypeStruct((N, D), jnp.float32),
  )(hist, acc, h, b)


def kernel(x, edge_index, W1, b1, W2, b2, W3, b3):
  src = edge_index[0].reshape(NW, 1, EPW)
  dst = edge_index[1].reshape(NW, 1, EPW)
  b1r = b1.reshape(1, D)
  b2r = b2.reshape(1, D)
  b3r = b3.reshape(1, D)

  ones_blk = jnp.ones((CHUNK, D), jnp.float32)
  zeros_rb = jnp.zeros((RB, D), jnp.float32)

  hist = _sc_histogram(dst, ones_blk, zeros_rb)
  h1 = _tc_first(hist, x, W1)
  acc1 = _sc_scatter(h1, src, dst, zeros_rb)
  h2 = _tc_advance(hist, acc1, h1, b1r, W2)
  acc2 = _sc_scatter(h2, src, dst, zeros_rb)
  h3 = _tc_advance(hist, acc2, h2, b2r, W3)
  acc3 = _sc_scatter(h3, src, dst, zeros_rb)
  out = _tc_final(hist, acc3, h3, b3r)
  return out
